# Initial kernel scaffold; baseline (speedup 1.0000x reference)
import jax
import jax.numpy as jnp
from jax.experimental import pallas as pl


def kernel(hidden_states, Wg, W1, b1, W2, b2):
    raise NotImplementedError("write your pallas kernel here")


# speedup vs baseline: 2.6141x; 2.6141x over previous
import functools

import jax
import jax.numpy as jnp
from jax import lax
from jax.experimental import pallas as pl
from jax.experimental.pallas import tpu as pltpu
from jax.experimental.pallas import tpu_sc as plsc

D_MODEL = 1024
D_FF = 2048
E = 64
TOP_K = 2
S = 2048

BLK = 128
NPAIR = S * TOP_K
NB = NPAIR // BLK + E
P_PAD = NB * BLK
TB = 256

_NW = 32
_RPW = P_PAD // _NW
_GCH = 64
_NCH = _RPW // _GCH
_TPW = S // _NW
_CCH = 32
_NCC = _TPW // _CCH


def _router_body(x_ref, wg_ref, i1_ref, i2_ref, g1_ref, g2_ref):
    logits = jnp.dot(x_ref[...], wg_ref[...], preferred_element_type=jnp.float32)
    iota = lax.broadcasted_iota(jnp.int32, logits.shape, 1)
    m1 = jnp.max(logits, axis=1, keepdims=True)
    i1 = jnp.min(jnp.where(logits == m1, iota, E), axis=1, keepdims=True)
    masked = jnp.where(iota == i1, -jnp.inf, logits)
    m2 = jnp.max(masked, axis=1, keepdims=True)
    i2 = jnp.min(jnp.where(masked == m2, iota, E), axis=1, keepdims=True)
    g1 = 1.0 / (1.0 + jnp.exp(m2 - m1))
    i1_ref[...] = i1
    i2_ref[...] = i2
    g1_ref[...] = g1
    g2_ref[...] = 1.0 - g1


def _router(x, wg):
    outs = [
        jax.ShapeDtypeStruct((S, 1), jnp.int32),
        jax.ShapeDtypeStruct((S, 1), jnp.int32),
        jax.ShapeDtypeStruct((S, 1), jnp.float32),
        jax.ShapeDtypeStruct((S, 1), jnp.float32),
    ]
    return pl.pallas_call(
        _router_body,
        grid=(S // TB,),
        in_specs=[
            pl.BlockSpec((TB, D_MODEL), lambda i: (i, 0)),
            pl.BlockSpec((D_MODEL, E), lambda i: (0, 0)),
        ],
        out_specs=[pl.BlockSpec((TB, 1), lambda i: (i, 0)) for _ in range(4)],
        out_shape=outs,
    )(x, wg)


def _sc_gather(x2d, idx3):
    mesh = plsc.VectorSubcoreMesh(core_axis_name="c", subcore_axis_name="s")

    @functools.partial(
        pl.kernel,
        mesh=mesh,
        out_type=jax.ShapeDtypeStruct((P_PAD, D_MODEL), jnp.float32),
        scratch_types=[
            pltpu.VMEM((_NCH, _GCH), jnp.int32),
            pltpu.VMEM((_GCH, D_MODEL), jnp.float32),
            pltpu.SemaphoreType.DMA,
        ],
    )
    def k(x_hbm, idx_hbm, out_hbm, idx_v, rows_v, sem):
        wid = lax.axis_index("s") * 2 + lax.axis_index("c")
        pltpu.sync_copy(idx_hbm.at[wid], idx_v)
        base = wid * _RPW
        for c in range(_NCH):
            pltpu.async_copy(x_hbm.at[idx_v.at[c]], rows_v, sem).wait()
            pltpu.sync_copy(rows_v, out_hbm.at[pl.ds(base + c * _GCH, _GCH)])

    return k(x2d, idx3)


def _ffn_body(b2e_ref, nact_ref, x_ref, w1_ref, b1_ref, w2_ref, b2_ref, g_ref,
              y_ref):
    pid = pl.program_id(0)

    @pl.when(pid < nact_ref[0])
    def _():
        h = jnp.dot(x_ref[...], w1_ref[0], preferred_element_type=jnp.float32)
        h = jax.nn.gelu(h + b1_ref[0])
        y = jnp.dot(h, w2_ref[0], preferred_element_type=jnp.float32)
        y = y + b2_ref[0]
        gb = jnp.broadcast_to(g_ref[0], (BLK, BLK))
        ri = lax.broadcasted_iota(jnp.int32, (BLK, BLK), 0)
        ci = lax.broadcasted_iota(jnp.int32, (BLK, BLK), 1)
        diag = jnp.where(ri == ci, gb, 0.0)
        y_ref[...] = jnp.dot(diag, y, preferred_element_type=jnp.float32)


def _ffn(x_pad, w1, b1, w2, b2, gates2d, b2e, nact):
    grid_spec = pltpu.PrefetchScalarGridSpec(
        num_scalar_prefetch=2,
        grid=(NB,),
        in_specs=[
            pl.BlockSpec((BLK, D_MODEL), lambda i, b2e, na: (i, 0)),
            pl.BlockSpec((1, D_MODEL, D_FF), lambda i, b2e, na: (b2e[i], 0, 0)),
            pl.BlockSpec((1, 1, D_FF), lambda i, b2e, na: (b2e[i], 0, 0)),
            pl.BlockSpec((1, D_FF, D_MODEL), lambda i, b2e, na: (b2e[i], 0, 0)),
            pl.BlockSpec((1, 1, D_MODEL), lambda i, b2e, na: (b2e[i], 0, 0)),
            pl.BlockSpec((1, 1, BLK), lambda i, b2e, na: (i, 0, 0)),
        ],
        out_specs=pl.BlockSpec((BLK, D_MODEL), lambda i, b2e, na: (i, 0)),
    )
    return pl.pallas_call(
        _ffn_body,
        grid_spec=grid_spec,
        out_shape=jax.ShapeDtypeStruct((P_PAD, D_MODEL), jnp.float32),
    )(b2e, nact, x_pad, w1, b1, w2, b2, gates2d)


def _sc_combine(y_pad, pa3, pb3):
    mesh = plsc.VectorSubcoreMesh(core_axis_name="c", subcore_axis_name="s")

    @functools.partial(
        pl.kernel,
        mesh=mesh,
        out_type=jax.ShapeDtypeStruct((S, D_MODEL), jnp.float32),
        scratch_types=[
            pltpu.VMEM((_NCC, _CCH), jnp.int32),
            pltpu.VMEM((_NCC, _CCH), jnp.int32),
            pltpu.VMEM((_CCH, D_MODEL), jnp.float32),
            pltpu.VMEM((_CCH, D_MODEL), jnp.float32),
            pltpu.SemaphoreType.DMA,
            pltpu.SemaphoreType.DMA,
        ],
    )
    def k(y_hbm, pa_hbm, pb_hbm, out_hbm, pa_v, pb_v, bufa, bufb, sema, semb):
        wid = lax.axis_index("s") * 2 + lax.axis_index("c")
        pltpu.sync_copy(pa_hbm.at[wid], pa_v)
        pltpu.sync_copy(pb_hbm.at[wid], pb_v)
        for c in range(_NCC):
            cpa = pltpu.async_copy(y_hbm.at[pa_v.at[c]], bufa, sema)
            cpb = pltpu.async_copy(y_hbm.at[pb_v.at[c]], bufb, semb)
            cpa.wait()
            cpb.wait()

            def body(r, carry):
                for q in range(D_MODEL // 16):
                    sl = pl.ds(q * 16, 16)
                    bufa[r, sl] = bufa[r, sl] + bufb[r, sl]
                return carry

            lax.fori_loop(0, _CCH, body, 0)
            pltpu.sync_copy(bufa, out_hbm.at[pl.ds(wid * _TPW + c * _CCH, _CCH)])

    return k(y_pad, pa3, pb3)


def _metadata(i1, i2, g1, g2):
    e_flat = jnp.stack([i1, i2], axis=1).reshape(-1)
    counts = jnp.zeros((E,), jnp.int32).at[e_flat].add(1)
    nblk_e = (counts + BLK - 1) // BLK
    blk_start = jnp.cumsum(nblk_e) - nblk_e
    nact = jnp.sum(nblk_e).astype(jnp.int32)
    row_off = blk_start * BLK

    order = jnp.argsort(e_flat).astype(jnp.int32)
    e_sorted = e_flat[order]
    grp_start = (jnp.cumsum(counts) - counts)[e_sorted]
    pos_sorted = (row_off[e_sorted]
                  + jnp.arange(NPAIR, dtype=jnp.int32) - grp_start)

    dest = jnp.zeros((NPAIR,), jnp.int32).at[order].set(pos_sorted)
    gather_tok = jnp.zeros((P_PAD,), jnp.int32).at[pos_sorted].set(order // 2)
    gates_flat = jnp.stack([g1, g2], axis=1).reshape(-1)
    gates_pad = jnp.zeros((P_PAD,), jnp.float32).at[pos_sorted].set(
        gates_flat[order])

    eids = jnp.arange(E, dtype=jnp.int32)
    tmp = jnp.zeros((NB,), jnp.int32).at[blk_start].max(
        jnp.where(nblk_e > 0, eids, 0))
    b2e = lax.cummax(tmp)
    return dest, gather_tok, gates_pad, b2e, nact.reshape(1)


def kernel(hidden_states, Wg, W1, b1, W2, b2):
    x = hidden_states.reshape(S, D_MODEL)
    i1, i2, g1, g2 = _router(x, Wg)
    i1, i2, g1, g2 = i1[:, 0], i2[:, 0], g1[:, 0], g2[:, 0]

    dest, gather_tok, gates_pad, b2e, nact = _metadata(i1, i2, g1, g2)

    x_pad = _sc_gather(x, gather_tok.reshape(_NW, _NCH, _GCH))
    y_pad = _ffn(x_pad, W1, b1.reshape(E, 1, D_FF), W2,
                 b2.reshape(E, 1, D_MODEL), gates_pad.reshape(NB, 1, BLK),
                 b2e, nact)

    dest2 = dest.reshape(S, TOP_K)
    pa3 = dest2[:, 0].reshape(_NW, _NCC, _CCH)
    pb3 = dest2[:, 1].reshape(_NW, _NCC, _CCH)
    out = _sc_combine(y_pad, pa3, pb3)
    return out.reshape(hidden_states.shape)


# speedup vs baseline: 4.4414x; 1.6990x over previous
import functools

import jax
import jax.numpy as jnp
from jax import lax
from jax.experimental import pallas as pl
from jax.experimental.pallas import tpu as pltpu
from jax.experimental.pallas import tpu_sc as plsc

D_MODEL = 1024
D_FF = 2048
E = 64
TOP_K = 2
S = 2048

BLK = 128
NPAIR = S * TOP_K
NB = NPAIR // BLK + E
P_PAD = NB * BLK
TB = 256
NTB = S // TB

_NW = 32
_TPW = S // _NW
_CCH = 32
_NCC = _TPW // _CCH


def _router_body(x_ref, wg_ref, i1_ref, i2_ref, g1_ref, g2_ref,
                 r1_ref, r2_ref, cnt_ref, base_ref):
    pid = pl.program_id(0)

    @pl.when(pid == 0)
    def _():
        base_ref[...] = jnp.zeros((1, E), jnp.float32)

    logits = jnp.dot(x_ref[...], wg_ref[...], preferred_element_type=jnp.float32)
    iota = lax.broadcasted_iota(jnp.int32, logits.shape, 1)
    m1 = jnp.max(logits, axis=1, keepdims=True)
    i1 = jnp.min(jnp.where(logits == m1, iota, E), axis=1, keepdims=True)
    masked = jnp.where(iota == i1, -jnp.inf, logits)
    m2 = jnp.max(masked, axis=1, keepdims=True)
    i2 = jnp.min(jnp.where(masked == m2, iota, E), axis=1, keepdims=True)
    g1 = 1.0 / (1.0 + jnp.exp(m2 - m1))
    i1_ref[...] = i1
    i2_ref[...] = i2
    g1_ref[...] = g1
    g2_ref[...] = 1.0 - g1

    oh1 = (iota == i1).astype(jnp.float32)
    oh2 = (iota == i2).astype(jnp.float32)
    ri = lax.broadcasted_iota(jnp.int32, (TB, TB), 0)
    ci = lax.broadcasted_iota(jnp.int32, (TB, TB), 1)
    ltri = (ri > ci).astype(jnp.float32)
    cum = jnp.dot(ltri, oh1 + oh2, preferred_element_type=jnp.float32)
    base = base_ref[...]
    r1 = jnp.sum(oh1 * (base + cum), axis=1, keepdims=True)
    r2 = jnp.sum(oh2 * (base + cum + oh1), axis=1, keepdims=True)
    r1_ref[...] = r1.astype(jnp.int32)
    r2_ref[...] = r2.astype(jnp.int32)
    newbase = base + jnp.sum(oh1 + oh2, axis=0, keepdims=True)
    base_ref[...] = newbase

    @pl.when(pid == NTB - 1)
    def _():
        cnt_ref[...] = newbase.astype(jnp.int32)


def _router(x, wg):
    outs = [
        jax.ShapeDtypeStruct((S, 1), jnp.int32),
        jax.ShapeDtypeStruct((S, 1), jnp.int32),
        jax.ShapeDtypeStruct((S, 1), jnp.float32),
        jax.ShapeDtypeStruct((S, 1), jnp.float32),
        jax.ShapeDtypeStruct((S, 1), jnp.int32),
        jax.ShapeDtypeStruct((S, 1), jnp.int32),
        jax.ShapeDtypeStruct((1, E), jnp.int32),
    ]
    tokspec = pl.BlockSpec((TB, 1), lambda i: (i, 0))
    return pl.pallas_call(
        _router_body,
        grid=(NTB,),
        in_specs=[
            pl.BlockSpec((TB, D_MODEL), lambda i: (i, 0)),
            pl.BlockSpec((D_MODEL, E), lambda i: (0, 0)),
        ],
        out_specs=[tokspec, tokspec, tokspec, tokspec, tokspec, tokspec,
                   pl.BlockSpec((1, E), lambda i: (0, 0))],
        out_shape=outs,
        scratch_shapes=[pltpu.VMEM((1, E), jnp.float32)],
    )(x, wg)


def _sc_dispatch(x2d, idx3):
    mesh = plsc.VectorSubcoreMesh(core_axis_name="c", subcore_axis_name="s")

    @functools.partial(
        pl.kernel,
        mesh=mesh,
        out_type=jax.ShapeDtypeStruct((P_PAD, D_MODEL), jnp.float32),
        scratch_types=[
            pltpu.VMEM((TOP_K, _TPW), jnp.int32),
            pltpu.VMEM((_TPW, D_MODEL), jnp.float32),
            pltpu.SemaphoreType.DMA,
        ],
    )
    def k(x_hbm, idx_hbm, out_hbm, idx_v, buf, sem):
        wid = lax.axis_index("s") * 2 + lax.axis_index("c")
        pltpu.sync_copy(idx_hbm.at[wid], idx_v)
        pltpu.sync_copy(x_hbm.at[pl.ds(wid * _TPW, _TPW)], buf)
        ca = pltpu.async_copy(buf, out_hbm.at[idx_v.at[0]], sem)
        cb = pltpu.async_copy(buf, out_hbm.at[idx_v.at[1]], sem)
        ca.wait()
        cb.wait()

    return k(x2d, idx3)


def _ffn_body(b2e_ref, nact_ref, x_ref, w1_ref, b1_ref, w2_ref, b2_ref, g_ref,
              y_ref):
    pid = pl.program_id(0)

    @pl.when(pid < nact_ref[0])
    def _():
        h = jnp.dot(x_ref[...], w1_ref[0], preferred_element_type=jnp.float32)
        h = jax.nn.gelu(h + b1_ref[0])
        y = jnp.dot(h, w2_ref[0], preferred_element_type=jnp.float32)
        y_ref[...] = (y + b2_ref[0]) * g_ref[...]


def _ffn(x_pad, w1, b1, w2, b2, gates_col, b2e, nact):
    grid_spec = pltpu.PrefetchScalarGridSpec(
        num_scalar_prefetch=2,
        grid=(NB,),
        in_specs=[
            pl.BlockSpec((BLK, D_MODEL), lambda i, b2e, na: (i, 0)),
            pl.BlockSpec((1, D_MODEL, D_FF), lambda i, b2e, na: (b2e[i], 0, 0)),
            pl.BlockSpec((1, 1, D_FF), lambda i, b2e, na: (b2e[i], 0, 0)),
            pl.BlockSpec((1, D_FF, D_MODEL), lambda i, b2e, na: (b2e[i], 0, 0)),
            pl.BlockSpec((1, 1, D_MODEL), lambda i, b2e, na: (b2e[i], 0, 0)),
            pl.BlockSpec((BLK, 1), lambda i, b2e, na: (i, 0)),
        ],
        out_specs=pl.BlockSpec((BLK, D_MODEL), lambda i, b2e, na: (i, 0)),
    )
    return pl.pallas_call(
        _ffn_body,
        grid_spec=grid_spec,
        out_shape=jax.ShapeDtypeStruct((P_PAD, D_MODEL), jnp.float32),
    )(b2e, nact, x_pad, w1, b1, w2, b2, gates_col)


def _sc_combine(y_pad, pa3, pb3):
    mesh = plsc.VectorSubcoreMesh(core_axis_name="c", subcore_axis_name="s")

    @functools.partial(
        pl.kernel,
        mesh=mesh,
        out_type=jax.ShapeDtypeStruct((S, D_MODEL), jnp.float32),
        scratch_types=[
            pltpu.VMEM((_NCC, _CCH), jnp.int32),
            pltpu.VMEM((_NCC, _CCH), jnp.int32),
            pltpu.VMEM((_CCH, D_MODEL), jnp.float32),
            pltpu.VMEM((_CCH, D_MODEL), jnp.float32),
            pltpu.SemaphoreType.DMA,
            pltpu.SemaphoreType.DMA,
        ],
    )
    def k(y_hbm, pa_hbm, pb_hbm, out_hbm, pa_v, pb_v, bufa, bufb, sema, semb):
        wid = lax.axis_index("s") * 2 + lax.axis_index("c")
        pltpu.sync_copy(pa_hbm.at[wid], pa_v)
        pltpu.sync_copy(pb_hbm.at[wid], pb_v)
        for c in range(_NCC):
            cpa = pltpu.async_copy(y_hbm.at[pa_v.at[c]], bufa, sema)
            cpb = pltpu.async_copy(y_hbm.at[pb_v.at[c]], bufb, semb)
            cpa.wait()
            cpb.wait()

            def body(r, carry):
                for q in range(D_MODEL // 16):
                    sl = pl.ds(q * 16, 16)
                    bufa[r, sl] = bufa[r, sl] + bufb[r, sl]
                return carry

            lax.fori_loop(0, _CCH, body, 0)
            pltpu.sync_copy(bufa, out_hbm.at[pl.ds(wid * _TPW + c * _CCH, _CCH)])

    return k(y_pad, pa3, pb3)


def kernel(hidden_states, Wg, W1, b1, W2, b2):
    x = hidden_states.reshape(S, D_MODEL)
    i1, i2, g1, g2, r1, r2, cnt = _router(x, Wg)
    i1, i2 = i1[:, 0], i2[:, 0]
    r1, r2 = r1[:, 0], r2[:, 0]
    counts = cnt[0]

    nblk_e = (counts + BLK - 1) // BLK
    blk_start = jnp.cumsum(nblk_e) - nblk_e
    nact = jnp.sum(nblk_e).astype(jnp.int32).reshape(1)
    row_off = blk_start * BLK

    eids = jnp.arange(E, dtype=jnp.int32)
    tmp = jnp.zeros((NB,), jnp.int32).at[blk_start].max(
        jnp.where(nblk_e > 0, eids, 0))
    b2e = lax.cummax(tmp)

    dest1 = row_off[i1] + r1
    dest2 = row_off[i2] + r2
    gates_pad = (jnp.zeros((P_PAD,), jnp.float32)
                 .at[dest1].set(g1[:, 0]).at[dest2].set(g2[:, 0]))

    idx3 = jnp.stack([dest1.reshape(_NW, _TPW), dest2.reshape(_NW, _TPW)],
                     axis=1)
    x_pad = _sc_dispatch(x, idx3)
    y_pad = _ffn(x_pad, W1, b1.reshape(E, 1, D_FF), W2,
                 b2.reshape(E, 1, D_MODEL), gates_pad.reshape(P_PAD, 1),
                 b2e, nact)

    pa3 = dest1.reshape(_NW, _NCC, _CCH)
    pb3 = dest2.reshape(_NW, _NCC, _CCH)
    out = _sc_combine(y_pad, pa3, pb3)
    return out.reshape(hidden_states.shape)


# speedup vs baseline: 5.9104x; 1.3308x over previous
import functools

import jax
import jax.numpy as jnp
from jax import lax
from jax.experimental import pallas as pl
from jax.experimental.pallas import tpu as pltpu
from jax.experimental.pallas import tpu_sc as plsc

D_MODEL = 1024
D_FF = 2048
E = 64
TOP_K = 2
S = 2048

BLK = 128
NPAIR = S * TOP_K
NB = NPAIR // BLK + E
NBP = 128
P_PAD = NB * BLK
TB = 256
NTB = S // TB

_NW = 32
_TPW = S // _NW
_CCH = 32
_NCC = _TPW // _CCH


def _router_body(x_ref, wg_ref, g1_ref, g2_ref, d1_ref, d2_ref,
                 b2e_ref, nact_ref, base_ref, i1s, i2s, r1s, r2s):
    pid = pl.program_id(0)

    @pl.when(pid == 0)
    def _():
        base_ref[...] = jnp.zeros((1, E), jnp.float32)

    logits = jnp.dot(x_ref[...], wg_ref[...], preferred_element_type=jnp.float32)
    iota = lax.broadcasted_iota(jnp.int32, logits.shape, 1)
    m1 = jnp.max(logits, axis=1, keepdims=True)
    i1 = jnp.min(jnp.where(logits == m1, iota, E), axis=1, keepdims=True)
    masked = jnp.where(iota == i1, -jnp.inf, logits)
    m2 = jnp.max(masked, axis=1, keepdims=True)
    i2 = jnp.min(jnp.where(masked == m2, iota, E), axis=1, keepdims=True)
    g1 = 1.0 / (1.0 + jnp.exp(m2 - m1))
    g1_ref[...] = jnp.broadcast_to(g1, (TB, 16))
    g2_ref[...] = jnp.broadcast_to(1.0 - g1, (TB, 16))

    oh1 = (iota == i1).astype(jnp.float32)
    oh2 = (iota == i2).astype(jnp.float32)
    ri = lax.broadcasted_iota(jnp.int32, (TB, TB), 0)
    ci = lax.broadcasted_iota(jnp.int32, (TB, TB), 1)
    ltri = (ri > ci).astype(jnp.float32)
    cum = jnp.dot(ltri, oh1 + oh2, preferred_element_type=jnp.float32)
    base = base_ref[...]
    r1 = jnp.sum(oh1 * (base + cum), axis=1, keepdims=True)
    r2 = jnp.sum(oh2 * (base + cum + oh1), axis=1, keepdims=True)
    sl = pl.ds(pid * TB, TB)
    i1s[sl, :] = i1
    i2s[sl, :] = i2
    r1s[sl, :] = r1
    r2s[sl, :] = r2
    newbase = base + jnp.sum(oh1 + oh2, axis=0, keepdims=True)
    base_ref[...] = newbase

    @pl.when(pid == NTB - 1)
    def _():
        counts = newbase
        nblk = jnp.floor((counts + (BLK - 1)) * (1.0 / BLK))
        ei = lax.broadcasted_iota(jnp.int32, (E, E), 0)
        ej = lax.broadcasted_iota(jnp.int32, (E, E), 1)
        incl = (ei <= ej).astype(jnp.float32)
        cum_incl = jnp.dot(nblk, incl, preferred_element_type=jnp.float32)
        blk_start = cum_incl - nblk
        row_off = blk_start * float(BLK)

        it = lax.broadcasted_iota(jnp.int32, (S, E), 1)
        sel1 = (it == i1s[...]).astype(jnp.float32)
        sel2 = (it == i2s[...]).astype(jnp.float32)
        d1 = jnp.sum(sel1 * row_off, axis=1, keepdims=True) + r1s[...]
        d2 = jnp.sum(sel2 * row_off, axis=1, keepdims=True) + r2s[...]
        d1_ref[...] = d1.astype(jnp.int32)
        d2_ref[...] = d2.astype(jnp.int32)

        bi = lax.broadcasted_iota(jnp.int32, (NBP, E), 0)
        be = lax.broadcasted_iota(jnp.int32, (NBP, E), 1)
        active = (bi >= blk_start.astype(jnp.int32)) & (nblk > 0.0)
        b2e_ref[...] = jnp.max(jnp.where(active, be, 0), axis=1, keepdims=True)
        nact_ref[...] = jnp.sum(nblk, axis=1, keepdims=True).astype(jnp.int32)


def _router(x, wg):
    outs = [
        jax.ShapeDtypeStruct((S, 16), jnp.float32),
        jax.ShapeDtypeStruct((S, 16), jnp.float32),
        jax.ShapeDtypeStruct((S, 1), jnp.int32),
        jax.ShapeDtypeStruct((S, 1), jnp.int32),
        jax.ShapeDtypeStruct((NBP, 1), jnp.int32),
        jax.ShapeDtypeStruct((1, 1), jnp.int32),
    ]
    tokspec = pl.BlockSpec((TB, 16), lambda i: (i, 0))
    whole = lambda i: (0, 0)
    return pl.pallas_call(
        _router_body,
        grid=(NTB,),
        in_specs=[
            pl.BlockSpec((TB, D_MODEL), lambda i: (i, 0)),
            pl.BlockSpec((D_MODEL, E), whole),
        ],
        out_specs=[tokspec, tokspec,
                   pl.BlockSpec((S, 1), whole), pl.BlockSpec((S, 1), whole),
                   pl.BlockSpec((NBP, 1), whole), pl.BlockSpec((1, 1), whole)],
        out_shape=outs,
        scratch_shapes=[pltpu.VMEM((1, E), jnp.float32),
                        pltpu.VMEM((S, 1), jnp.int32),
                        pltpu.VMEM((S, 1), jnp.int32),
                        pltpu.VMEM((S, 1), jnp.float32),
                        pltpu.VMEM((S, 1), jnp.float32)],
    )(x, wg)


def _sc_dispatch(x2d, d1w, d2w):
    mesh = plsc.VectorSubcoreMesh(core_axis_name="c", subcore_axis_name="s")

    @functools.partial(
        pl.kernel,
        mesh=mesh,
        out_type=jax.ShapeDtypeStruct((P_PAD, D_MODEL), jnp.float32),
        scratch_types=[
            pltpu.VMEM((TOP_K, _TPW), jnp.int32),
            pltpu.VMEM((_TPW, D_MODEL), jnp.float32),
            pltpu.SemaphoreType.DMA,
        ],
    )
    def k(x_hbm, d1_hbm, d2_hbm, out_hbm, idx_v, buf, sem):
        wid = lax.axis_index("s") * 2 + lax.axis_index("c")
        pltpu.sync_copy(d1_hbm.at[wid], idx_v.at[0])
        pltpu.sync_copy(d2_hbm.at[wid], idx_v.at[1])
        pltpu.sync_copy(x_hbm.at[pl.ds(wid * _TPW, _TPW)], buf)
        ca = pltpu.async_copy(buf, out_hbm.at[idx_v.at[0]], sem)
        cb = pltpu.async_copy(buf, out_hbm.at[idx_v.at[1]], sem)
        ca.wait()
        cb.wait()

    return k(x2d, d1w, d2w)


def _ffn_body(b2e_ref, nact_ref, x_ref, w1_ref, b1_ref, w2_ref, b2_ref, y_ref):
    pid = pl.program_id(0)

    @pl.when(pid < nact_ref[0])
    def _():
        h = jnp.dot(x_ref[...], w1_ref[0], preferred_element_type=jnp.float32)
        h = jax.nn.gelu(h + b1_ref[0])
        y = jnp.dot(h, w2_ref[0], preferred_element_type=jnp.float32)
        y_ref[...] = y + b2_ref[0]


def _ffn(x_pad, w1, b1, w2, b2, b2e, nact):
    grid_spec = pltpu.PrefetchScalarGridSpec(
        num_scalar_prefetch=2,
        grid=(NB,),
        in_specs=[
            pl.BlockSpec((BLK, D_MODEL), lambda i, b2e, na: (i, 0)),
            pl.BlockSpec((1, D_MODEL, D_FF), lambda i, b2e, na: (b2e[i], 0, 0)),
            pl.BlockSpec((1, 1, D_FF), lambda i, b2e, na: (b2e[i], 0, 0)),
            pl.BlockSpec((1, D_FF, D_MODEL), lambda i, b2e, na: (b2e[i], 0, 0)),
            pl.BlockSpec((1, 1, D_MODEL), lambda i, b2e, na: (b2e[i], 0, 0)),
        ],
        out_specs=pl.BlockSpec((BLK, D_MODEL), lambda i, b2e, na: (i, 0)),
    )
    return pl.pallas_call(
        _ffn_body,
        grid_spec=grid_spec,
        out_shape=jax.ShapeDtypeStruct((P_PAD, D_MODEL), jnp.float32),
    )(b2e, nact, x_pad, w1, b1, w2, b2)


def _sc_combine(y_pad, d1w, d2w, g1w, g2w):
    mesh = plsc.VectorSubcoreMesh(core_axis_name="c", subcore_axis_name="s")

    @functools.partial(
        pl.kernel,
        mesh=mesh,
        out_type=jax.ShapeDtypeStruct((S, D_MODEL), jnp.float32),
        scratch_types=[
            pltpu.VMEM((_TPW,), jnp.int32),
            pltpu.VMEM((_TPW,), jnp.int32),
            pltpu.VMEM((_TPW, 16), jnp.float32),
            pltpu.VMEM((_TPW, 16), jnp.float32),
            pltpu.VMEM((_CCH, D_MODEL), jnp.float32),
            pltpu.VMEM((_CCH, D_MODEL), jnp.float32),
            pltpu.SemaphoreType.DMA,
            pltpu.SemaphoreType.DMA,
        ],
    )
    def k(y_hbm, d1_hbm, d2_hbm, g1_hbm, g2_hbm, out_hbm,
          pa_v, pb_v, ga_v, gb_v, bufa, bufb, sema, semb):
        wid = lax.axis_index("s") * 2 + lax.axis_index("c")
        pltpu.sync_copy(d1_hbm.at[wid], pa_v)
        pltpu.sync_copy(d2_hbm.at[wid], pb_v)
        pltpu.sync_copy(g1_hbm.at[wid], ga_v)
        pltpu.sync_copy(g2_hbm.at[wid], gb_v)
        for c in range(_NCC):
            cpa = pltpu.async_copy(
                y_hbm.at[pa_v.at[pl.ds(c * _CCH, _CCH)]], bufa, sema)
            cpb = pltpu.async_copy(
                y_hbm.at[pb_v.at[pl.ds(c * _CCH, _CCH)]], bufb, semb)
            cpa.wait()
            cpb.wait()

            def body(r, carry):
                tok = c * _CCH + r
                ga = ga_v[tok, :]
                gb = gb_v[tok, :]
                for q in range(D_MODEL // 16):
                    sl = pl.ds(q * 16, 16)
                    bufa[r, sl] = ga * bufa[r, sl] + gb * bufb[r, sl]
                return carry

            lax.fori_loop(0, _CCH, body, 0)
            pltpu.sync_copy(bufa, out_hbm.at[pl.ds(wid * _TPW + c * _CCH, _CCH)])

    return k(y_pad, d1w, d2w, g1w, g2w)


def kernel(hidden_states, Wg, W1, b1, W2, b2):
    x = hidden_states.reshape(S, D_MODEL)
    g1, g2, dest1, dest2, b2e, nact = _router(x, Wg)

    d1w = dest1.reshape(_NW, _TPW)
    d2w = dest2.reshape(_NW, _TPW)
    x_pad = _sc_dispatch(x, d1w, d2w)
    y_pad = _ffn(x_pad, W1, b1.reshape(E, 1, D_FF), W2,
                 b2.reshape(E, 1, D_MODEL), b2e.reshape(NBP), nact.reshape(1))

    out = _sc_combine(y_pad, d1w, d2w,
                      g1.reshape(_NW, _TPW, 16), g2.reshape(_NW, _TPW, 16))
    return out.reshape(hidden_states.shape)


# speedup vs baseline: 6.1990x; 1.0488x over previous
import functools

import jax
import jax.numpy as jnp
from jax import lax
from jax.experimental import pallas as pl
from jax.experimental.pallas import tpu as pltpu
from jax.experimental.pallas import tpu_sc as plsc

D_MODEL = 1024
D_FF = 2048
E = 64
TOP_K = 2
S = 2048

BLK = 128
NPAIR = S * TOP_K
NB = NPAIR // BLK + E
NBP = 128
P_PAD = NB * BLK
TB = 256
NTB = S // TB

_NW = 32
_TPW = S // _NW
_CCH = 16
_NCC = _TPW // _CCH


def _router_body(x_ref, wg_ref, g1_ref, g2_ref, d1_ref, d2_ref,
                 b2e_ref, nact_ref, base_ref, i1s, i2s, r1s, r2s):
    pid = pl.program_id(0)

    @pl.when(pid == 0)
    def _():
        base_ref[...] = jnp.zeros((1, E), jnp.float32)

    logits = jnp.dot(x_ref[...], wg_ref[...], preferred_element_type=jnp.float32)
    iota = lax.broadcasted_iota(jnp.int32, logits.shape, 1)
    m1 = jnp.max(logits, axis=1, keepdims=True)
    i1 = jnp.min(jnp.where(logits == m1, iota, E), axis=1, keepdims=True)
    masked = jnp.where(iota == i1, -jnp.inf, logits)
    m2 = jnp.max(masked, axis=1, keepdims=True)
    i2 = jnp.min(jnp.where(masked == m2, iota, E), axis=1, keepdims=True)
    g1 = 1.0 / (1.0 + jnp.exp(m2 - m1))
    g1_ref[...] = jnp.broadcast_to(g1, (TB, 16))
    g2_ref[...] = jnp.broadcast_to(1.0 - g1, (TB, 16))

    oh1 = (iota == i1).astype(jnp.float32)
    oh2 = (iota == i2).astype(jnp.float32)
    ri = lax.broadcasted_iota(jnp.int32, (TB, TB), 0)
    ci = lax.broadcasted_iota(jnp.int32, (TB, TB), 1)
    ltri = (ri > ci).astype(jnp.float32)
    cum = jnp.dot(ltri, oh1 + oh2, preferred_element_type=jnp.float32)
    base = base_ref[...]
    r1 = jnp.sum(oh1 * (base + cum), axis=1, keepdims=True)
    r2 = jnp.sum(oh2 * (base + cum + oh1), axis=1, keepdims=True)
    sl = pl.ds(pid * TB, TB)
    i1s[sl, :] = i1
    i2s[sl, :] = i2
    r1s[sl, :] = r1
    r2s[sl, :] = r2
    newbase = base + jnp.sum(oh1 + oh2, axis=0, keepdims=True)
    base_ref[...] = newbase

    @pl.when(pid == NTB - 1)
    def _():
        counts = newbase
        nblk = jnp.floor((counts + (BLK - 1)) * (1.0 / BLK))
        ei = lax.broadcasted_iota(jnp.int32, (E, E), 0)
        ej = lax.broadcasted_iota(jnp.int32, (E, E), 1)
        incl = (ei <= ej).astype(jnp.float32)
        cum_incl = jnp.dot(nblk, incl, preferred_element_type=jnp.float32)
        blk_start = cum_incl - nblk
        row_off = blk_start * float(BLK)

        it = lax.broadcasted_iota(jnp.int32, (S, E), 1)
        sel1 = (it == i1s[...]).astype(jnp.float32)
        sel2 = (it == i2s[...]).astype(jnp.float32)
        d1 = jnp.sum(sel1 * row_off, axis=1, keepdims=True) + r1s[...]
        d2 = jnp.sum(sel2 * row_off, axis=1, keepdims=True) + r2s[...]
        d1_ref[...] = d1.astype(jnp.int32)
        d2_ref[...] = d2.astype(jnp.int32)

        bi = lax.broadcasted_iota(jnp.int32, (NBP, E), 0)
        be = lax.broadcasted_iota(jnp.int32, (NBP, E), 1)
        active = (bi >= blk_start.astype(jnp.int32)) & (nblk > 0.0)
        b2e_ref[...] = jnp.max(jnp.where(active, be, 0), axis=1, keepdims=True)
        nact_ref[...] = jnp.sum(nblk, axis=1, keepdims=True).astype(jnp.int32)


def _router(x, wg):
    outs = [
        jax.ShapeDtypeStruct((S, 16), jnp.float32),
        jax.ShapeDtypeStruct((S, 16), jnp.float32),
        jax.ShapeDtypeStruct((S, 1), jnp.int32),
        jax.ShapeDtypeStruct((S, 1), jnp.int32),
        jax.ShapeDtypeStruct((NBP, 1), jnp.int32),
        jax.ShapeDtypeStruct((1, 1), jnp.int32),
    ]
    tokspec = pl.BlockSpec((TB, 16), lambda i: (i, 0))
    whole = lambda i: (0, 0)
    return pl.pallas_call(
        _router_body,
        grid=(NTB,),
        in_specs=[
            pl.BlockSpec((TB, D_MODEL), lambda i: (i, 0)),
            pl.BlockSpec((D_MODEL, E), whole),
        ],
        out_specs=[tokspec, tokspec,
                   pl.BlockSpec((S, 1), whole), pl.BlockSpec((S, 1), whole),
                   pl.BlockSpec((NBP, 1), whole), pl.BlockSpec((1, 1), whole)],
        out_shape=outs,
        scratch_shapes=[pltpu.VMEM((1, E), jnp.float32),
                        pltpu.VMEM((S, 1), jnp.int32),
                        pltpu.VMEM((S, 1), jnp.int32),
                        pltpu.VMEM((S, 1), jnp.float32),
                        pltpu.VMEM((S, 1), jnp.float32)],
    )(x, wg)


def _sc_dispatch(x2d, d1w, d2w):
    mesh = plsc.VectorSubcoreMesh(core_axis_name="c", subcore_axis_name="s")

    @functools.partial(
        pl.kernel,
        mesh=mesh,
        out_type=jax.ShapeDtypeStruct((P_PAD, D_MODEL), jnp.float32),
        scratch_types=[
            pltpu.VMEM((TOP_K, _TPW), jnp.int32),
            pltpu.VMEM((_TPW, D_MODEL), jnp.float32),
            pltpu.SemaphoreType.DMA,
        ],
    )
    def k(x_hbm, d1_hbm, d2_hbm, out_hbm, idx_v, buf, sem):
        wid = lax.axis_index("s") * 2 + lax.axis_index("c")
        pltpu.sync_copy(d1_hbm.at[wid], idx_v.at[0])
        pltpu.sync_copy(d2_hbm.at[wid], idx_v.at[1])
        pltpu.sync_copy(x_hbm.at[pl.ds(wid * _TPW, _TPW)], buf)
        ca = pltpu.async_copy(buf, out_hbm.at[idx_v.at[0]], sem)
        cb = pltpu.async_copy(buf, out_hbm.at[idx_v.at[1]], sem)
        ca.wait()
        cb.wait()

    return k(x2d, d1w, d2w)


def _ffn_body(b2e_ref, nact_ref, x_ref, w1_ref, b1_ref, w2_ref, b2_ref, y_ref):
    pid = pl.program_id(0)

    @pl.when(pid < nact_ref[0])
    def _():
        h = jnp.dot(x_ref[...], w1_ref[0], preferred_element_type=jnp.float32)
        h = jax.nn.gelu(h + b1_ref[0])
        y = jnp.dot(h, w2_ref[0], preferred_element_type=jnp.float32)
        y_ref[...] = y + b2_ref[0]


def _ffn(x_pad, w1, b1, w2, b2, b2e, nact):
    clamp = lambda i, na: jnp.where(i < na[0], i, na[0] - 1)
    grid_spec = pltpu.PrefetchScalarGridSpec(
        num_scalar_prefetch=2,
        grid=(NB,),
        in_specs=[
            pl.BlockSpec((BLK, D_MODEL),
                         lambda i, b2e, na: (clamp(i, na), 0)),
            pl.BlockSpec((1, D_MODEL, D_FF), lambda i, b2e, na: (b2e[i], 0, 0)),
            pl.BlockSpec((1, 1, D_FF), lambda i, b2e, na: (b2e[i], 0, 0)),
            pl.BlockSpec((1, D_FF, D_MODEL), lambda i, b2e, na: (b2e[i], 0, 0)),
            pl.BlockSpec((1, 1, D_MODEL), lambda i, b2e, na: (b2e[i], 0, 0)),
        ],
        out_specs=pl.BlockSpec((BLK, D_MODEL),
                               lambda i, b2e, na: (clamp(i, na), 0)),
    )
    return pl.pallas_call(
        _ffn_body,
        grid_spec=grid_spec,
        out_shape=jax.ShapeDtypeStruct((P_PAD, D_MODEL), jnp.float32),
    )(b2e, nact, x_pad, w1, b1, w2, b2)


def _sc_combine(y_pad, d1w, d2w, g1w, g2w):
    mesh = plsc.VectorSubcoreMesh(core_axis_name="c", subcore_axis_name="s")

    @functools.partial(
        pl.kernel,
        mesh=mesh,
        out_type=jax.ShapeDtypeStruct((S, D_MODEL), jnp.float32),
        scratch_types=[
            pltpu.VMEM((_TPW,), jnp.int32),
            pltpu.VMEM((_TPW,), jnp.int32),
            pltpu.VMEM((_TPW, 16), jnp.float32),
            pltpu.VMEM((_TPW, 16), jnp.float32),
            pltpu.VMEM((2, _CCH, D_MODEL), jnp.float32),
            pltpu.VMEM((2, _CCH, D_MODEL), jnp.float32),
            pltpu.VMEM((2, _CCH, D_MODEL), jnp.float32),
            pltpu.SemaphoreType.DMA,
            pltpu.SemaphoreType.DMA,
            pltpu.SemaphoreType.DMA,
            pltpu.SemaphoreType.DMA,
            pltpu.SemaphoreType.DMA,
            pltpu.SemaphoreType.DMA,
        ],
    )
    def k(y_hbm, d1_hbm, d2_hbm, g1_hbm, g2_hbm, out_hbm,
          pa_v, pb_v, ga_v, gb_v, bufa, bufb, bufo,
          sga0, sga1, sgb0, sgb1, swr0, swr1):
        sga = [sga0, sga1]
        sgb = [sgb0, sgb1]
        swr = [swr0, swr1]
        wid = lax.axis_index("s") * 2 + lax.axis_index("c")
        pltpu.sync_copy(d1_hbm.at[wid], pa_v)
        pltpu.sync_copy(d2_hbm.at[wid], pb_v)
        pltpu.sync_copy(g1_hbm.at[wid], ga_v)
        pltpu.sync_copy(g2_hbm.at[wid], gb_v)

        def gathers(c, s):
            cpa = pltpu.async_copy(
                y_hbm.at[pa_v.at[pl.ds(c * _CCH, _CCH)]], bufa.at[s], sga[s])
            cpb = pltpu.async_copy(
                y_hbm.at[pb_v.at[pl.ds(c * _CCH, _CCH)]], bufb.at[s], sgb[s])
            return cpa, cpb

        pend = [gathers(0, 0), gathers(1, 1)]
        wr = [None, None]
        for c in range(_NCC):
            s = c % 2
            cpa, cpb = pend[s]
            cpa.wait()
            cpb.wait()
            if wr[s] is not None:
                wr[s].wait()

            def body(r, carry, c=c, s=s):
                tok = c * _CCH + r
                ga = ga_v[tok, :]
                gb = gb_v[tok, :]
                for q in range(D_MODEL // 16):
                    sl = pl.ds(q * 16, 16)
                    bufo[s, r, sl] = ga * bufa[s, r, sl] + gb * bufb[s, r, sl]
                return carry

            lax.fori_loop(0, _CCH, body, 0)
            if c + 2 < _NCC:
                pend[s] = gathers(c + 2, s)
            wr[s] = pltpu.async_copy(
                bufo.at[s], out_hbm.at[pl.ds(wid * _TPW + c * _CCH, _CCH)],
                swr[s])
        for s in range(2):
            if wr[s] is not None:
                wr[s].wait()

    return k(y_pad, d1w, d2w, g1w, g2w)


def kernel(hidden_states, Wg, W1, b1, W2, b2):
    x = hidden_states.reshape(S, D_MODEL)
    g1, g2, dest1, dest2, b2e, nact = _router(x, Wg)

    d1w = dest1.reshape(_NW, _TPW)
    d2w = dest2.reshape(_NW, _TPW)
    x_pad = _sc_dispatch(x, d1w, d2w)
    y_pad = _ffn(x_pad, W1, b1.reshape(E, 1, D_FF), W2,
                 b2.reshape(E, 1, D_MODEL), b2e.reshape(NBP), nact.reshape(1))

    out = _sc_combine(y_pad, d1w, d2w,
                      g1.reshape(_NW, _TPW, 16), g2.reshape(_NW, _TPW, 16))
    return out.reshape(hidden_states.shape)


# speedup vs baseline: 6.2310x; 1.0052x over previous
import functools

import jax
import jax.numpy as jnp
from jax import lax
from jax.experimental import pallas as pl
from jax.experimental.pallas import tpu as pltpu
from jax.experimental.pallas import tpu_sc as plsc

D_MODEL = 1024
D_FF = 2048
E = 64
TOP_K = 2
S = 2048

BLK = 128
NPAIR = S * TOP_K
NB = NPAIR // BLK + E
NBP = 128
P_PAD = NB * BLK
TB = 512
NTB = S // TB

_NW = 32
_TPW = S // _NW
_CCH = 16
_NCC = _TPW // _CCH


def _router_body(x_ref, wg_ref, g1_ref, g2_ref, d1_ref, d2_ref,
                 b2e_ref, nact_ref, base_ref, i1s, i2s, r1s, r2s):
    pid = pl.program_id(0)

    @pl.when(pid == 0)
    def _():
        base_ref[...] = jnp.zeros((1, E), jnp.float32)

    logits = jnp.dot(x_ref[...], wg_ref[...], preferred_element_type=jnp.float32)
    iota = lax.broadcasted_iota(jnp.int32, logits.shape, 1)
    m1 = jnp.max(logits, axis=1, keepdims=True)
    i1 = jnp.min(jnp.where(logits == m1, iota, E), axis=1, keepdims=True)
    masked = jnp.where(iota == i1, -jnp.inf, logits)
    m2 = jnp.max(masked, axis=1, keepdims=True)
    i2 = jnp.min(jnp.where(masked == m2, iota, E), axis=1, keepdims=True)
    g1 = 1.0 / (1.0 + jnp.exp(m2 - m1))
    g1_ref[...] = jnp.broadcast_to(g1, (TB, 16))
    g2_ref[...] = jnp.broadcast_to(1.0 - g1, (TB, 16))

    oh1 = (iota == i1).astype(jnp.float32)
    oh2 = (iota == i2).astype(jnp.float32)
    ri = lax.broadcasted_iota(jnp.int32, (TB, TB), 0)
    ci = lax.broadcasted_iota(jnp.int32, (TB, TB), 1)
    ltri = (ri > ci).astype(jnp.float32)
    cum = jnp.dot(ltri, oh1 + oh2, preferred_element_type=jnp.float32)
    base = base_ref[...]
    r1 = jnp.sum(oh1 * (base + cum), axis=1, keepdims=True)
    r2 = jnp.sum(oh2 * (base + cum + oh1), axis=1, keepdims=True)
    sl = pl.ds(pid * TB, TB)
    i1s[sl, :] = i1
    i2s[sl, :] = i2
    r1s[sl, :] = r1
    r2s[sl, :] = r2
    newbase = base + jnp.sum(oh1 + oh2, axis=0, keepdims=True)
    base_ref[...] = newbase

    @pl.when(pid == NTB - 1)
    def _():
        counts = newbase
        nblk = jnp.floor((counts + (BLK - 1)) * (1.0 / BLK))
        ei = lax.broadcasted_iota(jnp.int32, (E, E), 0)
        ej = lax.broadcasted_iota(jnp.int32, (E, E), 1)
        incl = (ei <= ej).astype(jnp.float32)
        cum_incl = jnp.dot(nblk, incl, preferred_element_type=jnp.float32)
        blk_start = cum_incl - nblk
        row_off = blk_start * float(BLK)

        it = lax.broadcasted_iota(jnp.int32, (S, E), 1)
        sel1 = (it == i1s[...]).astype(jnp.float32)
        sel2 = (it == i2s[...]).astype(jnp.float32)
        d1 = jnp.sum(sel1 * row_off, axis=1, keepdims=True) + r1s[...]
        d2 = jnp.sum(sel2 * row_off, axis=1, keepdims=True) + r2s[...]
        d1_ref[...] = d1.astype(jnp.int32)
        d2_ref[...] = d2.astype(jnp.int32)

        bi = lax.broadcasted_iota(jnp.int32, (NBP, E), 0)
        be = lax.broadcasted_iota(jnp.int32, (NBP, E), 1)
        active = (bi >= blk_start.astype(jnp.int32)) & (nblk > 0.0)
        b2e_ref[...] = jnp.max(jnp.where(active, be, 0), axis=1, keepdims=True)
        nact_ref[...] = jnp.sum(nblk, axis=1, keepdims=True).astype(jnp.int32)


def _router(x, wg):
    outs = [
        jax.ShapeDtypeStruct((S, 16), jnp.float32),
        jax.ShapeDtypeStruct((S, 16), jnp.float32),
        jax.ShapeDtypeStruct((S, 1), jnp.int32),
        jax.ShapeDtypeStruct((S, 1), jnp.int32),
        jax.ShapeDtypeStruct((NBP, 1), jnp.int32),
        jax.ShapeDtypeStruct((1, 1), jnp.int32),
    ]
    tokspec = pl.BlockSpec((TB, 16), lambda i: (i, 0))
    whole = lambda i: (0, 0)
    return pl.pallas_call(
        _router_body,
        grid=(NTB,),
        in_specs=[
            pl.BlockSpec((TB, D_MODEL), lambda i: (i, 0)),
            pl.BlockSpec((D_MODEL, E), whole),
        ],
        out_specs=[tokspec, tokspec,
                   pl.BlockSpec((S, 1), whole), pl.BlockSpec((S, 1), whole),
                   pl.BlockSpec((NBP, 1), whole), pl.BlockSpec((1, 1), whole)],
        out_shape=outs,
        scratch_shapes=[pltpu.VMEM((1, E), jnp.float32),
                        pltpu.VMEM((S, 1), jnp.int32),
                        pltpu.VMEM((S, 1), jnp.int32),
                        pltpu.VMEM((S, 1), jnp.float32),
                        pltpu.VMEM((S, 1), jnp.float32)],
    )(x, wg)


def _sc_dispatch(x2d, d1w, d2w):
    mesh = plsc.VectorSubcoreMesh(core_axis_name="c", subcore_axis_name="s")

    @functools.partial(
        pl.kernel,
        mesh=mesh,
        out_type=jax.ShapeDtypeStruct((P_PAD, D_MODEL), jnp.float32),
        scratch_types=[
            pltpu.VMEM((TOP_K, _TPW), jnp.int32),
            pltpu.VMEM((_TPW, D_MODEL), jnp.float32),
            pltpu.SemaphoreType.DMA,
        ],
    )
    def k(x_hbm, d1_hbm, d2_hbm, out_hbm, idx_v, buf, sem):
        wid = lax.axis_index("s") * 2 + lax.axis_index("c")
        pltpu.sync_copy(d1_hbm.at[wid], idx_v.at[0])
        pltpu.sync_copy(d2_hbm.at[wid], idx_v.at[1])
        pltpu.sync_copy(x_hbm.at[pl.ds(wid * _TPW, _TPW)], buf)
        ca = pltpu.async_copy(buf, out_hbm.at[idx_v.at[0]], sem)
        cb = pltpu.async_copy(buf, out_hbm.at[idx_v.at[1]], sem)
        ca.wait()
        cb.wait()

    return k(x2d, d1w, d2w)


def _ffn_body(b2e_ref, nact_ref, x_ref, w1_ref, b1_ref, w2_ref, b2_ref, y_ref):
    pid = pl.program_id(0)

    @pl.when(pid < nact_ref[0])
    def _():
        h = jnp.dot(x_ref[...], w1_ref[0], preferred_element_type=jnp.float32)
        h = jax.nn.gelu(h + b1_ref[0])
        y = jnp.dot(h, w2_ref[0], preferred_element_type=jnp.float32)
        y_ref[...] = y + b2_ref[0]


def _ffn(x_pad, w1, b1, w2, b2, b2e, nact):
    clamp = lambda i, na: jnp.where(i < na[0], i, na[0] - 1)
    grid_spec = pltpu.PrefetchScalarGridSpec(
        num_scalar_prefetch=2,
        grid=(NB,),
        in_specs=[
            pl.BlockSpec((BLK, D_MODEL),
                         lambda i, b2e, na: (clamp(i, na), 0)),
            pl.BlockSpec((1, D_MODEL, D_FF), lambda i, b2e, na: (b2e[i], 0, 0)),
            pl.BlockSpec((1, 1, D_FF), lambda i, b2e, na: (b2e[i], 0, 0)),
            pl.BlockSpec((1, D_FF, D_MODEL), lambda i, b2e, na: (b2e[i], 0, 0)),
            pl.BlockSpec((1, 1, D_MODEL), lambda i, b2e, na: (b2e[i], 0, 0)),
        ],
        out_specs=pl.BlockSpec((BLK, D_MODEL),
                               lambda i, b2e, na: (clamp(i, na), 0)),
    )
    return pl.pallas_call(
        _ffn_body,
        grid_spec=grid_spec,
        out_shape=jax.ShapeDtypeStruct((P_PAD, D_MODEL), jnp.float32),
    )(b2e, nact, x_pad, w1, b1, w2, b2)


def _sc_combine(y_pad, d1w, d2w, g1w, g2w):
    mesh = plsc.VectorSubcoreMesh(core_axis_name="c", subcore_axis_name="s")

    @functools.partial(
        pl.kernel,
        mesh=mesh,
        out_type=jax.ShapeDtypeStruct((S, D_MODEL), jnp.float32),
        scratch_types=[
            pltpu.VMEM((_TPW,), jnp.int32),
            pltpu.VMEM((_TPW,), jnp.int32),
            pltpu.VMEM((_TPW, 16), jnp.float32),
            pltpu.VMEM((_TPW, 16), jnp.float32),
            pltpu.VMEM((2, _CCH, D_MODEL), jnp.float32),
            pltpu.VMEM((2, _CCH, D_MODEL), jnp.float32),
            pltpu.VMEM((2, _CCH, D_MODEL), jnp.float32),
            pltpu.SemaphoreType.DMA,
            pltpu.SemaphoreType.DMA,
            pltpu.SemaphoreType.DMA,
            pltpu.SemaphoreType.DMA,
            pltpu.SemaphoreType.DMA,
            pltpu.SemaphoreType.DMA,
        ],
    )
    def k(y_hbm, d1_hbm, d2_hbm, g1_hbm, g2_hbm, out_hbm,
          pa_v, pb_v, ga_v, gb_v, bufa, bufb, bufo,
          sga0, sga1, sgb0, sgb1, swr0, swr1):
        sga = [sga0, sga1]
        sgb = [sgb0, sgb1]
        swr = [swr0, swr1]
        wid = lax.axis_index("s") * 2 + lax.axis_index("c")
        pltpu.sync_copy(d1_hbm.at[wid], pa_v)
        pltpu.sync_copy(d2_hbm.at[wid], pb_v)
        pltpu.sync_copy(g1_hbm.at[wid], ga_v)
        pltpu.sync_copy(g2_hbm.at[wid], gb_v)

        def gathers(c, s):
            cpa = pltpu.async_copy(
                y_hbm.at[pa_v.at[pl.ds(c * _CCH, _CCH)]], bufa.at[s], sga[s])
            cpb = pltpu.async_copy(
                y_hbm.at[pb_v.at[pl.ds(c * _CCH, _CCH)]], bufb.at[s], sgb[s])
            return cpa, cpb

        pend = [gathers(0, 0), gathers(1, 1)]
        wr = [None, None]
        for c in range(_NCC):
            s = c % 2
            cpa, cpb = pend[s]
            cpa.wait()
            cpb.wait()
            if wr[s] is not None:
                wr[s].wait()

            def body(r, carry, c=c, s=s):
                tok = c * _CCH + r
                ga = ga_v[tok, :]
                gb = gb_v[tok, :]
                for q in range(D_MODEL // 16):
                    sl = pl.ds(q * 16, 16)
                    bufo[s, r, sl] = ga * bufa[s, r, sl] + gb * bufb[s, r, sl]
                return carry

            lax.fori_loop(0, _CCH, body, 0)
            if c + 2 < _NCC:
                pend[s] = gathers(c + 2, s)
            wr[s] = pltpu.async_copy(
                bufo.at[s], out_hbm.at[pl.ds(wid * _TPW + c * _CCH, _CCH)],
                swr[s])
        for s in range(2):
            if wr[s] is not None:
                wr[s].wait()

    return k(y_pad, d1w, d2w, g1w, g2w)


def kernel(hidden_states, Wg, W1, b1, W2, b2):
    x = hidden_states.reshape(S, D_MODEL)
    g1, g2, dest1, dest2, b2e, nact = _router(x, Wg)

    d1w = dest1.reshape(_NW, _TPW)
    d2w = dest2.reshape(_NW, _TPW)
    x_pad = _sc_dispatch(x, d1w, d2w)
    y_pad = _ffn(x_pad, W1, b1.reshape(E, 1, D_FF), W2,
                 b2.reshape(E, 1, D_MODEL), b2e.reshape(NBP), nact.reshape(1))

    out = _sc_combine(y_pad, d1w, d2w,
                      g1.reshape(_NW, _TPW, 16), g2.reshape(_NW, _TPW, 16))
    return out.reshape(hidden_states.shape)
